# baseline (speedup 1.0000x reference)
import jax
import jax.numpy as jnp
from jax import lax
from jax.experimental import pallas as pl
from jax.experimental.pallas import tpu as pltpu
from jax.experimental.pallas import tpu_sc as plsc

_NC = 2
_NS = 16
_K = 125
_CPP = 40
_NPAD = 10240
_WCHUNKS = tuple((j * 80, 80) for j in range(8))


def _mm_body(x_ref, w_ref, b_ref, o_ref):
    o_ref[...] = lax.dot_general(
        x_ref[...], w_ref[...],
        dimension_numbers=(((1,), (1,)), ((), ())),
        preferred_element_type=jnp.float32) + b_ref[...]


def _linear(x, w, b):
    n, d_in = x.shape
    d_out = w.shape[0]
    blk = 1000
    return pl.pallas_call(
        _mm_body,
        grid=(n // blk,),
        in_specs=[
            pl.BlockSpec((blk, d_in), lambda i: (i, 0)),
            pl.BlockSpec((d_out, d_in), lambda i: (0, 0)),
            pl.BlockSpec((1, d_out), lambda i: (0, 0)),
        ],
        out_specs=pl.BlockSpec((blk, d_out), lambda i: (i, 0)),
        out_shape=jax.ShapeDtypeStruct((n, d_out), jnp.float32),
    )(x, w, b.reshape(1, d_out))


def _sc_body(h_hbm, ei_hbm, zero_hbm, p_hbm,
             acc, rowv, colv, gbuf0, gbuf1, sem0, sem1):
    c = lax.axis_index("c")
    s = lax.axis_index("s")
    wid = c * _NS + s
    n_rows = acc.shape[0]
    slab = n_rows // _NS
    chunks = rowv.shape[0]
    npass = ei_hbm.shape[2] // chunks

    pltpu.sync_copy(zero_hbm, acc.at[pl.ds(s * slab, slab)])
    plsc.subcore_barrier()

    for p in range(npass):
        pltpu.sync_copy(ei_hbm.at[0, wid, pl.ds(p * chunks, chunks)], rowv)
        pltpu.sync_copy(ei_hbm.at[1, wid, pl.ds(p * chunks, chunks)], colv)
        pltpu.async_copy(h_hbm.at[colv.at[0]], gbuf0, sem0)

        def pair(io, carry):
            i = io * 2
            pltpu.async_copy(h_hbm.at[colv.at[i + 1]], gbuf1, sem1)
            pltpu.make_async_copy(h_hbm.at[colv.at[i]], gbuf0, sem0).wait()
            pltpu.sync_copy(gbuf0, acc.at[rowv.at[i]], add=True)

            @pl.when(i + 2 < chunks)
            def _():
                pltpu.async_copy(h_hbm.at[colv.at[i + 2]], gbuf0, sem0)
            pltpu.make_async_copy(
                h_hbm.at[colv.at[i + 1]], gbuf1, sem1).wait()
            pltpu.sync_copy(gbuf1, acc.at[rowv.at[i + 1]], add=True)
            return carry
        lax.fori_loop(0, chunks // 2, pair, 0)
    plsc.subcore_barrier()

    pltpu.sync_copy(acc.at[pl.ds(s * slab, slab)],
                    p_hbm.at[pl.ds(c * n_rows + s * slab, slab)])


def _aggregate(h, edge_index):
    n, d = h.shape
    e = edge_index.shape[1]
    nw = _NC * _NS
    npass = -(-e // (_K * nw * _CPP))
    cpw = npass * _CPP
    pad = cpw * _K * nw - e
    if pad:
        trash = n + jnp.arange(pad, dtype=jnp.int32) % (_NPAD - n)
        pad_rc = jnp.stack([trash, jnp.zeros((pad,), jnp.int32)])
        edge_index = jnp.concatenate([edge_index, pad_rc], axis=1)
    ei = edge_index.reshape(2, nw, cpw, _K)
    zeros = jnp.zeros((_NPAD // _NS, d), jnp.float32)
    mesh = plsc.VectorSubcoreMesh(core_axis_name="c", subcore_axis_name="s",
                                  num_cores=_NC, num_subcores=_NS)
    f = pl.kernel(
        _sc_body,
        out_type=jax.ShapeDtypeStruct((_NC * _NPAD, d), jnp.float32),
        mesh=mesh,
        scratch_types=[
            pltpu.VMEM_SHARED((_NPAD, d), jnp.float32),
            pltpu.VMEM((_CPP, _K), jnp.int32),
            pltpu.VMEM((_CPP, _K), jnp.int32),
            pltpu.VMEM((_K, d), jnp.float32),
            pltpu.VMEM((_K, d), jnp.float32),
            pltpu.SemaphoreType.DMA,
            pltpu.SemaphoreType.DMA,
        ],
    )
    return f(h, ei, zeros)


def _add_body(a_ref, b_ref, o_ref):
    o_ref[...] = a_ref[...] + b_ref[...]


def _combine(p, n, d):
    blk = _NPAD // 8
    g = -(-n // blk)
    off = _NPAD // blk
    return pl.pallas_call(
        _add_body,
        grid=(g,),
        in_specs=[pl.BlockSpec((blk, d), lambda i: (i, 0)),
                  pl.BlockSpec((blk, d), lambda i: (i + off, 0))],
        out_specs=pl.BlockSpec((blk, d), lambda i: (i, 0)),
        out_shape=jax.ShapeDtypeStruct((n, d), jnp.float32),
    )(p, p)


def kernel(x, edge_index, W, b):
    h = _linear(x, W, b)
    p = _aggregate(h, edge_index)
    n, d = h.shape
    return _combine(p, n, d)


# speedup vs baseline: 1.0056x; 1.0056x over previous
import jax
import jax.numpy as jnp
from jax import lax
from jax.experimental import pallas as pl
from jax.experimental.pallas import tpu as pltpu
from jax.experimental.pallas import tpu_sc as plsc

_NC = 2
_NS = 16
_K = 125
_CPP = 40
_NPAD = 10240
_WCHUNKS = tuple((j * 80, 80) for j in range(8))


def _mm_body(x_ref, w_ref, b_ref, o_ref):
    o_ref[...] = lax.dot_general(
        x_ref[...], w_ref[...],
        dimension_numbers=(((1,), (1,)), ((), ())),
        preferred_element_type=jnp.float32) + b_ref[...]


def _linear(x, w, b):
    n, d_in = x.shape
    d_out = w.shape[0]
    blk = 1000
    return pl.pallas_call(
        _mm_body,
        grid=(n // blk,),
        in_specs=[
            pl.BlockSpec((blk, d_in), lambda i: (i, 0)),
            pl.BlockSpec((d_out, d_in), lambda i: (0, 0)),
            pl.BlockSpec((1, d_out), lambda i: (0, 0)),
        ],
        out_specs=pl.BlockSpec((blk, d_out), lambda i: (i, 0)),
        out_shape=jax.ShapeDtypeStruct((n, d_out), jnp.float32),
    )(x, w, b.reshape(1, d_out))


def _sc_body(h_hbm, ei_hbm, zero_hbm, p_hbm,
             acc, rowv, colv, gbuf0, gbuf1, sem0, sem1):
    c = lax.axis_index("c")
    s = lax.axis_index("s")
    wid = c * _NS + s
    n_rows = acc.shape[0]
    slab = n_rows // _NS
    chunks = rowv.shape[0]
    npass = ei_hbm.shape[2] // chunks

    zbuf = gbuf0.at[pl.ds(0, zero_hbm.shape[0])]
    pltpu.sync_copy(zero_hbm, zbuf)
    for r0, nr in _WCHUNKS:
        pltpu.sync_copy(gbuf0.at[pl.ds(0, nr)],
                        acc.at[pl.ds(s * slab + r0, nr)])
    plsc.subcore_barrier()

    for p in range(npass):
        pltpu.sync_copy(ei_hbm.at[0, wid, pl.ds(p * chunks, chunks)], rowv)
        pltpu.sync_copy(ei_hbm.at[1, wid, pl.ds(p * chunks, chunks)], colv)
        pltpu.async_copy(h_hbm.at[colv.at[0]], gbuf0, sem0)

        def pair(io, carry):
            i = io * 2
            pltpu.async_copy(h_hbm.at[colv.at[i + 1]], gbuf1, sem1)
            pltpu.make_async_copy(h_hbm.at[colv.at[i]], gbuf0, sem0).wait()
            pltpu.sync_copy(gbuf0, acc.at[rowv.at[i]], add=True)

            @pl.when(i + 2 < chunks)
            def _():
                pltpu.async_copy(h_hbm.at[colv.at[i + 2]], gbuf0, sem0)
            pltpu.make_async_copy(
                h_hbm.at[colv.at[i + 1]], gbuf1, sem1).wait()
            pltpu.sync_copy(gbuf1, acc.at[rowv.at[i + 1]], add=True)
            return carry
        lax.fori_loop(0, chunks // 2, pair, 0)
    plsc.subcore_barrier()

    for r0, nr in _WCHUNKS:
        pltpu.sync_copy(acc.at[pl.ds(s * slab + r0, nr)],
                        gbuf0.at[pl.ds(0, nr)])
        pltpu.sync_copy(gbuf0.at[pl.ds(0, nr)],
                        p_hbm.at[pl.ds(c * n_rows + s * slab + r0, nr)])


def _aggregate(h, edge_index):
    n, d = h.shape
    e = edge_index.shape[1]
    nw = _NC * _NS
    npass = -(-e // (_K * nw * _CPP))
    cpw = npass * _CPP
    pad = cpw * _K * nw - e
    if pad:
        trash = n + jnp.arange(pad, dtype=jnp.int32) % (_NPAD - n)
        pad_rc = jnp.stack([trash, jnp.zeros((pad,), jnp.int32)])
        edge_index = jnp.concatenate([edge_index, pad_rc], axis=1)
    ei = edge_index.reshape(2, nw, cpw, _K)
    zeros = jnp.zeros((80, d), jnp.float32)
    mesh = plsc.VectorSubcoreMesh(core_axis_name="c", subcore_axis_name="s",
                                  num_cores=_NC, num_subcores=_NS)
    f = pl.kernel(
        _sc_body,
        out_type=jax.ShapeDtypeStruct((_NC * _NPAD, d), jnp.float32),
        mesh=mesh,
        scratch_types=[
            pltpu.VMEM_SHARED((_NPAD, d), jnp.float32),
            pltpu.VMEM((_CPP, _K), jnp.int32),
            pltpu.VMEM((_CPP, _K), jnp.int32),
            pltpu.VMEM((_K, d), jnp.float32),
            pltpu.VMEM((_K, d), jnp.float32),
            pltpu.SemaphoreType.DMA,
            pltpu.SemaphoreType.DMA,
        ],
    )
    return f(h, ei, zeros)


def _add_body(a_ref, b_ref, o_ref):
    o_ref[...] = a_ref[...] + b_ref[...]


def _combine(p, n, d):
    blk = _NPAD // 8
    g = -(-n // blk)
    off = _NPAD // blk
    return pl.pallas_call(
        _add_body,
        grid=(g,),
        in_specs=[pl.BlockSpec((blk, d), lambda i: (i, 0)),
                  pl.BlockSpec((blk, d), lambda i: (i + off, 0))],
        out_specs=pl.BlockSpec((blk, d), lambda i: (i, 0)),
        out_shape=jax.ShapeDtypeStruct((n, d), jnp.float32),
    )(p, p)


def kernel(x, edge_index, W, b):
    h = _linear(x, W, b)
    p = _aggregate(h, edge_index)
    n, d = h.shape
    return _combine(p, n, d)
